# baseline (speedup 1.0000x reference)
import functools

import jax
import jax.numpy as jnp
from jax import lax
from jax.experimental import pallas as pl
from jax.experimental.pallas import tpu as pltpu
from jax.experimental.pallas import tpu_sc as plsc

BATCH = 16384
EMBED_DIM = 32
VOCAB_ROWS = 100001
CHUNK = 2048
NCHUNK = BATCH // CHUNK
LANES = 16


def _gather_kernel(
    table_hbm,
    idx_hbm,
    out_hbm,
    row_v,
    idx_a,
    idx_b,
    out_a,
    out_b,
    sem_row,
    sem_idx,
    sem_out,
):
    num_cores = plsc.get_sparse_core_info().num_cores
    w = lax.axis_index("s") * num_cores + lax.axis_index("c")
    idx_bufs = [idx_a, idx_b]
    out_bufs = [out_a, out_b]

    row_copy = pltpu.async_copy(table_hbm.at[w], row_v, sem_row)
    idx_copies = [
        pltpu.async_copy(
            idx_hbm.at[pl.ds(c * CHUNK, CHUNK)], idx_bufs[c], sem_idx.at[c]
        )
        for c in range(2)
    ]
    row_copy.wait()

    out_copies = [None, None]
    for c in range(NCHUNK):
        idx_copies[c % 2].wait()
        if c + 2 < NCHUNK:
            idx_copies[c % 2] = pltpu.async_copy(
                idx_hbm.at[pl.ds((c + 2) * CHUNK, CHUNK)],
                idx_bufs[c % 2],
                sem_idx.at[c % 2],
            )
        if out_copies[c % 2] is not None:
            out_copies[c % 2].wait()

        idx_ref = idx_bufs[c % 2]
        out_ref = out_bufs[c % 2]

        @plsc.parallel_loop(0, CHUNK, LANES, unroll=8)
        def _(i):
            idx = idx_ref[pl.ds(i, LANES)]
            out_ref[pl.ds(i, LANES)] = plsc.load_gather(row_v, [idx])

        out_copies[c % 2] = pltpu.async_copy(
            out_bufs[c % 2], out_hbm.at[w, pl.ds(c * CHUNK, CHUNK)], sem_out.at[c % 2]
        )
    for c in range(2):
        out_copies[c].wait()


def kernel(inputs, table):
    mesh = plsc.VectorSubcoreMesh(core_axis_name="c", subcore_axis_name="s")
    run = functools.partial(
        pl.kernel,
        mesh=mesh,
        out_type=jax.ShapeDtypeStruct((EMBED_DIM, BATCH), jnp.float32),
        scratch_types=[
            pltpu.VMEM((VOCAB_ROWS,), jnp.float32),
            pltpu.VMEM((CHUNK,), jnp.int32),
            pltpu.VMEM((CHUNK,), jnp.int32),
            pltpu.VMEM((CHUNK,), jnp.float32),
            pltpu.VMEM((CHUNK,), jnp.float32),
            pltpu.SemaphoreType.DMA,
            pltpu.SemaphoreType.DMA((2,)),
            pltpu.SemaphoreType.DMA((2,)),
        ],
        compiler_params=pltpu.CompilerParams(needs_layout_passes=False),
    )(_gather_kernel)
    out_t = run(table.T, inputs.astype(jnp.int32))
    return out_t.T


# speedup vs baseline: 1.0601x; 1.0601x over previous
import functools

import jax
import jax.numpy as jnp
from jax import lax
from jax.experimental import pallas as pl
from jax.experimental.pallas import tpu as pltpu
from jax.experimental.pallas import tpu_sc as plsc

BATCH = 16384
EMBED_DIM = 32
VOCAB_ROWS = 100001
CHUNK = 4096
NCHUNK = BATCH // CHUNK
LANES = 16


def _gather_kernel(
    table_hbm,
    idx_hbm,
    out_hbm,
    row_v,
    idx_a,
    idx_b,
    out_a,
    out_b,
    sem_row,
    sem_idx,
    sem_out,
):
    num_cores = plsc.get_sparse_core_info().num_cores
    w = lax.axis_index("s") * num_cores + lax.axis_index("c")
    idx_bufs = [idx_a, idx_b]
    out_bufs = [out_a, out_b]

    row_copy = pltpu.async_copy(table_hbm.at[w], row_v, sem_row)
    idx_copies = [
        pltpu.async_copy(
            idx_hbm.at[pl.ds(c * CHUNK, CHUNK)], idx_bufs[c], sem_idx.at[c]
        )
        for c in range(2)
    ]
    row_copy.wait()

    out_copies = [None, None]
    for c in range(NCHUNK):
        idx_copies[c % 2].wait()
        if c + 2 < NCHUNK:
            idx_copies[c % 2] = pltpu.async_copy(
                idx_hbm.at[pl.ds((c + 2) * CHUNK, CHUNK)],
                idx_bufs[c % 2],
                sem_idx.at[c % 2],
            )
        if out_copies[c % 2] is not None:
            out_copies[c % 2].wait()

        idx_ref = idx_bufs[c % 2]
        out_ref = out_bufs[c % 2]

        @plsc.parallel_loop(0, CHUNK, LANES, unroll=8)
        def _(i):
            idx = idx_ref[pl.ds(i, LANES)]
            out_ref[pl.ds(i, LANES)] = plsc.load_gather(row_v, [idx])

        out_copies[c % 2] = pltpu.async_copy(
            out_bufs[c % 2], out_hbm.at[w, pl.ds(c * CHUNK, CHUNK)], sem_out.at[c % 2]
        )
    for c in range(2):
        out_copies[c].wait()


def kernel(inputs, table):
    mesh = plsc.VectorSubcoreMesh(core_axis_name="c", subcore_axis_name="s")
    run = functools.partial(
        pl.kernel,
        mesh=mesh,
        out_type=jax.ShapeDtypeStruct((EMBED_DIM, BATCH), jnp.float32),
        scratch_types=[
            pltpu.VMEM((VOCAB_ROWS,), jnp.float32),
            pltpu.VMEM((CHUNK,), jnp.int32),
            pltpu.VMEM((CHUNK,), jnp.int32),
            pltpu.VMEM((CHUNK,), jnp.float32),
            pltpu.VMEM((CHUNK,), jnp.float32),
            pltpu.SemaphoreType.DMA,
            pltpu.SemaphoreType.DMA((2,)),
            pltpu.SemaphoreType.DMA((2,)),
        ],
        compiler_params=pltpu.CompilerParams(
            needs_layout_passes=False, skip_device_barrier=True
        ),
    )(_gather_kernel)
    out_t = run(table.T, inputs.astype(jnp.int32))
    return out_t.T


# speedup vs baseline: 1.0605x; 1.0004x over previous
import functools

import jax
import jax.numpy as jnp
from jax import lax
from jax.experimental import pallas as pl
from jax.experimental.pallas import tpu as pltpu
from jax.experimental.pallas import tpu_sc as plsc

BATCH = 16384
EMBED_DIM = 32
VOCAB_ROWS = 100001
CHUNK = 4096
NCHUNK = BATCH // CHUNK
LANES = 16


def _gather_kernel(
    table_hbm,
    idx_hbm,
    out_hbm,
    row_v,
    idx_a,
    idx_b,
    out_a,
    out_b,
    sem_row,
    sem_idx,
    sem_out,
):
    num_cores = plsc.get_sparse_core_info().num_cores
    w = lax.axis_index("s") * num_cores + lax.axis_index("c")
    idx_bufs = [idx_a, idx_b]
    out_bufs = [out_a, out_b]

    row_copy = pltpu.async_copy(table_hbm.at[w], row_v, sem_row)
    idx_copies = [
        pltpu.async_copy(
            idx_hbm.at[pl.ds(c * CHUNK, CHUNK)], idx_bufs[c], sem_idx.at[c]
        )
        for c in range(2)
    ]
    row_copy.wait()

    out_copies = [None, None]
    for c in range(NCHUNK):
        idx_copies[c % 2].wait()
        if c + 2 < NCHUNK:
            idx_copies[c % 2] = pltpu.async_copy(
                idx_hbm.at[pl.ds((c + 2) * CHUNK, CHUNK)],
                idx_bufs[c % 2],
                sem_idx.at[c % 2],
            )
        if out_copies[c % 2] is not None:
            out_copies[c % 2].wait()

        idx_ref = idx_bufs[c % 2]
        out_ref = out_bufs[c % 2]

        @plsc.parallel_loop(0, CHUNK, LANES, unroll=8)
        def _(i):
            idx = idx_ref[pl.ds(i, LANES)]
            out_ref[pl.ds(i, LANES)] = plsc.load_gather(row_v, [idx])

        out_copies[c % 2] = pltpu.async_copy(
            out_bufs[c % 2], out_hbm.at[w, pl.ds(c * CHUNK, CHUNK)], sem_out.at[c % 2]
        )
    for c in range(2):
        out_copies[c].wait()


def kernel(inputs, table):
    mesh = plsc.VectorSubcoreMesh(core_axis_name="c", subcore_axis_name="s")
    run = functools.partial(
        pl.kernel,
        mesh=mesh,
        out_type=jax.ShapeDtypeStruct((EMBED_DIM, BATCH), jnp.float32),
        scratch_types=[
            pltpu.VMEM((VOCAB_ROWS,), jnp.float32),
            pltpu.VMEM((CHUNK,), jnp.int32),
            pltpu.VMEM((CHUNK,), jnp.int32),
            pltpu.VMEM((CHUNK,), jnp.float32),
            pltpu.VMEM((CHUNK,), jnp.float32),
            pltpu.SemaphoreType.DMA,
            pltpu.SemaphoreType.DMA((2,)),
            pltpu.SemaphoreType.DMA((2,)),
        ],
        compiler_params=pltpu.CompilerParams(needs_layout_passes=False),
    )(_gather_kernel)
    out_t = run(table.T, inputs.astype(jnp.int32))
    return out_t.T


# speedup vs baseline: 1.1293x; 1.0649x over previous
import functools

import jax
import jax.numpy as jnp
from jax import lax
from jax.experimental import pallas as pl
from jax.experimental.pallas import tpu as pltpu
from jax.experimental.pallas import tpu_sc as plsc

BATCH = 16384
EMBED_DIM = 32
VOCAB_ROWS = 100001
CHUNK = 4096
NCHUNK = BATCH // CHUNK
LANES = 16


def _gather_kernel(
    table_hbm,
    idx_hbm,
    out_hbm,
    row_v,
    idx_a,
    idx_b,
    idx_c,
    idx_d,
    out_a,
    out_b,
    sem_row,
    sem_idx,
    sem_out,
):
    num_cores = plsc.get_sparse_core_info().num_cores
    w = lax.axis_index("s") * num_cores + lax.axis_index("c")
    idx_bufs = [idx_a, idx_b, idx_c, idx_d]
    out_bufs = [out_a, out_b]

    row_copy = pltpu.async_copy(table_hbm.at[w], row_v, sem_row)
    idx_copies = [
        pltpu.async_copy(
            idx_hbm.at[pl.ds(c * CHUNK, CHUNK)], idx_bufs[c], sem_idx.at[c]
        )
        for c in range(NCHUNK)
    ]
    row_copy.wait()

    out_copies = [None, None]
    for c in range(NCHUNK):
        idx_copies[c].wait()
        if out_copies[c % 2] is not None:
            out_copies[c % 2].wait()

        idx_ref = idx_bufs[c]
        out_ref = out_bufs[c % 2]

        @plsc.parallel_loop(0, CHUNK, LANES, unroll=8)
        def _(i):
            idx = idx_ref[pl.ds(i, LANES)]
            out_ref[pl.ds(i, LANES)] = plsc.load_gather(row_v, [idx])

        out_copies[c % 2] = pltpu.async_copy(
            out_bufs[c % 2], out_hbm.at[w, pl.ds(c * CHUNK, CHUNK)], sem_out.at[c % 2]
        )
    for c in range(2):
        out_copies[c].wait()


def kernel(inputs, table):
    mesh = plsc.VectorSubcoreMesh(core_axis_name="c", subcore_axis_name="s")
    run = functools.partial(
        pl.kernel,
        mesh=mesh,
        out_type=jax.ShapeDtypeStruct((EMBED_DIM, BATCH), jnp.float32),
        scratch_types=[
            pltpu.VMEM((VOCAB_ROWS,), jnp.float32),
            pltpu.VMEM((CHUNK,), jnp.int32),
            pltpu.VMEM((CHUNK,), jnp.int32),
            pltpu.VMEM((CHUNK,), jnp.int32),
            pltpu.VMEM((CHUNK,), jnp.int32),
            pltpu.VMEM((CHUNK,), jnp.float32),
            pltpu.VMEM((CHUNK,), jnp.float32),
            pltpu.SemaphoreType.DMA,
            pltpu.SemaphoreType.DMA((NCHUNK,)),
            pltpu.SemaphoreType.DMA((2,)),
        ],
        compiler_params=pltpu.CompilerParams(needs_layout_passes=False),
    )(_gather_kernel)
    out_t = run(table.T, inputs.astype(jnp.int32))
    return out_t.T
